# baseline (speedup 1.0000x reference)
import functools

import jax
import jax.numpy as jnp
from jax import lax
from jax.experimental import pallas as pl
from jax.experimental.pallas import tpu as pltpu
from jax.experimental.pallas import tpu_sc as plsc

BATCH = 16384
EMBED = 64
FEAT = 128
HID = 128

NC = 2
NS = 16
NW = NC * NS
BPW = BATCH // NW
CHUNK = 128
NCH = BPW // CHUNK
IDX_ROWS = BATCH // CHUNK


def _gather_body(uidx, pidx, utab, ptab, u_out, p_out,
                 uidx_v, pidx_v, urows_v, prows_v, sem):
    wid = lax.axis_index("s") * NC + lax.axis_index("c")
    row0 = wid * NCH
    base = wid * BPW
    pltpu.sync_copy(uidx.at[pl.ds(row0, NCH)], uidx_v)
    pltpu.sync_copy(pidx.at[pl.ds(row0, NCH)], pidx_v)
    copies = []
    for j in range(NCH):
        copies.append(pltpu.async_copy(
            utab.at[uidx_v.at[j]], urows_v.at[pl.ds(j * CHUNK, CHUNK)], sem))
        copies.append(pltpu.async_copy(
            ptab.at[pidx_v.at[j]], prows_v.at[pl.ds(j * CHUNK, CHUNK)], sem))
    for c in copies:
        c.wait()
    pltpu.sync_copy(urows_v, u_out.at[pl.ds(base, BPW)])
    pltpu.sync_copy(prows_v, p_out.at[pl.ds(base, BPW)])


def _sc_gather(uidx, pidx, utab, ptab):
    mesh = plsc.VectorSubcoreMesh(core_axis_name="c", subcore_axis_name="s")
    fn = functools.partial(
        pl.kernel,
        mesh=mesh,
        compiler_params=pltpu.CompilerParams(use_tc_tiling_on_sc=False),
        out_type=(
            jax.ShapeDtypeStruct((BATCH, EMBED), jnp.float32),
            jax.ShapeDtypeStruct((BATCH, EMBED), jnp.float32),
        ),
        scratch_types=[
            pltpu.VMEM((NCH, CHUNK), jnp.int32),
            pltpu.VMEM((NCH, CHUNK), jnp.int32),
            pltpu.VMEM((BPW, EMBED), jnp.float32),
            pltpu.VMEM((BPW, EMBED), jnp.float32),
            pltpu.SemaphoreType.DMA,
        ],
    )(_gather_body)
    return fn(uidx, pidx, utab, ptab)


def _mlp_body(u, p, f, w1u, w1p, w1f, b1, w2, b2, w3t, b3, o):
    x1 = jnp.dot(u[:], w1u[:], preferred_element_type=jnp.float32)
    x1 = x1 + jnp.dot(p[:], w1p[:], preferred_element_type=jnp.float32)
    x1 = x1 + jnp.dot(f[:], w1f[:], preferred_element_type=jnp.float32)
    h1 = jnp.maximum(x1 + b1[:], 0.0)
    h2 = jnp.maximum(
        jnp.dot(h1, w2[:], preferred_element_type=jnp.float32) + b2[:], 0.0)
    s = jnp.sum(h2 * w3t[:], axis=1, keepdims=True) + b3[:]
    o[:] = 1.0 / (1.0 + jnp.exp(-s))


def _tc_mlp(u, p, f, w1u, w1p, w1f, b1, w2, b2, w3t, b3, tile=512):
    grid = BATCH // tile
    full = lambda i: (0, 0)
    return pl.pallas_call(
        _mlp_body,
        grid=(grid,),
        in_specs=[
            pl.BlockSpec((tile, EMBED), lambda i: (i, 0)),
            pl.BlockSpec((tile, EMBED), lambda i: (i, 0)),
            pl.BlockSpec((tile, FEAT), lambda i: (i, 0)),
            pl.BlockSpec((EMBED, HID), full),
            pl.BlockSpec((EMBED, HID), full),
            pl.BlockSpec((FEAT, HID), full),
            pl.BlockSpec((1, HID), full),
            pl.BlockSpec((HID, HID), full),
            pl.BlockSpec((1, HID), full),
            pl.BlockSpec((1, HID), full),
            pl.BlockSpec((1, 1), full),
        ],
        out_specs=pl.BlockSpec((tile, 1), lambda i: (i, 0)),
        out_shape=jax.ShapeDtypeStruct((BATCH, 1), jnp.float32),
    )(u, p, f, w1u, w1p, w1f, b1, w2, b2, w3t, b3)


def kernel(user_indices, post_indices, features, user_table, post_table,
           W1, b1, W2, b2, W3, b3):
    ui = user_indices.astype(jnp.int32).reshape(IDX_ROWS, CHUNK)
    pi = post_indices.astype(jnp.int32).reshape(IDX_ROWS, CHUNK)
    u, p = _sc_gather(ui, pi, user_table, post_table)
    o = _tc_mlp(
        u, p, features,
        W1[:EMBED], W1[EMBED:2 * EMBED], W1[2 * EMBED:],
        b1.reshape(1, HID), W2, b2.reshape(1, HID),
        W3.reshape(1, HID), b3.reshape(1, 1))
    return o.reshape(BATCH)
